# baseline (speedup 1.0000x reference)
import functools

import jax
import jax.numpy as jnp
from jax import lax
from jax.experimental import pallas as pl
from jax.experimental.pallas import tpu as pltpu
from jax.experimental.pallas import tpu_sc as plsc

_N = 100000
_E = 6400000
_NC = 2
_NS = 16
_LN2 = 0.6931471805599453

_NPAD = 100352
_SEG = _NPAD // _NS
_CHUNK = 4096
_CHUNKS_PER_W = 50
_EPAD = _NC * _NS * _CHUNKS_PER_W * _CHUNK

_EDGE_BUF = [
    ("sidx", jnp.int32), ("ridx", jnp.int32),
    ("bsx", jnp.float32), ("bsy", jnp.float32), ("bsz", jnp.float32),
    ("gsx", jnp.float32), ("gsy", jnp.float32), ("gsz", jnp.float32),
    ("grx", jnp.float32), ("gry", jnp.float32), ("grz", jnp.float32),
    ("ev", jnp.float32),
]


def _sc_energies(snd, rcv, shx, shy, shz, px, py, pz, zeros):
    mesh = plsc.VectorSubcoreMesh(core_axis_name="c", subcore_axis_name="s")

    scratch = [pltpu.VMEM_SHARED((_NPAD,), jnp.float32)] * 4
    scratch += [pltpu.VMEM((_CHUNK,), dt) for _, dt in _EDGE_BUF] * 2
    scratch += [pltpu.SemaphoreType.DMA] * 6

    @functools.partial(
        pl.kernel,
        out_type=jax.ShapeDtypeStruct((_NC, _NPAD), jnp.float32),
        mesh=mesh,
        scratch_types=scratch,
    )
    def k(snd_r, rcv_r, shx_r, shy_r, shz_r, px_r, py_r, pz_r, z_r, out,
          spx, spy, spz, acc, *bufs_and_sems):
        nb = len(_EDGE_BUF)
        bufa = dict(zip([n for n, _ in _EDGE_BUF], bufs_and_sems[:nb]))
        bufb = dict(zip([n for n, _ in _EDGE_BUF], bufs_and_sems[nb:2 * nb]))
        (sem_ld_a, sem_ld_b, sem_g_a, sem_g_b,
         sem_sc_a, sem_sc_b) = bufs_and_sems[2 * nb:]

        cid = lax.axis_index("c")
        sid = lax.axis_index("s")
        seg = pl.ds(sid * _SEG, _SEG)
        pltpu.sync_copy(px_r.at[seg], spx.at[seg])
        pltpu.sync_copy(py_r.at[seg], spy.at[seg])
        pltpu.sync_copy(pz_r.at[seg], spz.at[seg])
        pltpu.sync_copy(z_r.at[seg], acc.at[seg])
        plsc.subcore_barrier()

        w = cid * _NS + sid
        base = w * _CHUNKS_PER_W

        def load_pairs(kk, b):
            es = pl.ds((base + kk) * _CHUNK, _CHUNK)
            return [(snd_r.at[es], b["sidx"]), (rcv_r.at[es], b["ridx"]),
                    (shx_r.at[es], b["bsx"]), (shy_r.at[es], b["bsy"]),
                    (shz_r.at[es], b["bsz"])]

        def gather_pairs(b):
            return [(spx.at[b["sidx"]], b["gsx"]),
                    (spy.at[b["sidx"]], b["gsy"]),
                    (spz.at[b["sidx"]], b["gsz"]),
                    (spx.at[b["ridx"]], b["grx"]),
                    (spy.at[b["ridx"]], b["gry"]),
                    (spz.at[b["ridx"]], b["grz"])]

        def issue(pairs, sem, add=False):
            for src, dst in pairs:
                pltpu.async_copy(src, dst, sem, add=add)

        def drain(pairs, sem):
            for src, dst in pairs:
                pltpu.make_async_copy(src, dst, sem).wait()

        def compute(b):
            gsx, gsy, gsz = b["gsx"], b["gsy"], b["gsz"]
            grx, gry, grz = b["grx"], b["gry"], b["grz"]
            bsx, bsy, bsz = b["bsx"], b["bsy"], b["bsz"]
            ev = b["ev"]

            def group_body(j, jcarry):
                cs = pl.ds(j * 16, 16)
                dx = grx[cs] - gsx[cs] + bsx[cs]
                dy = gry[cs] - gsy[cs] + bsy[cs]
                dz = grz[cs] - gsz[cs] + bsz[cs]
                d2 = jnp.maximum(dx * dx + dy * dy + dz * dz,
                                 jnp.float32(1e-35))
                bits = lax.bitcast_convert_type(d2, jnp.int32)
                rsq = lax.bitcast_convert_type(0x5F3759DF - (bits >> 1),
                                               jnp.float32)
                hd2 = 0.5 * d2
                rsq = rsq * (1.5 - hd2 * rsq * rsq)
                rsq = rsq * (1.5 - hd2 * rsq * rsq)
                rsq = rsq * (1.5 - hd2 * rsq * rsq)
                ln = d2 * rsq
                ev[cs] = jnp.exp(-2.0 * ln - _LN2)
                return jcarry

            lax.fori_loop(0, _CHUNK // 16, group_body, 0)

        def scat_pair(b):
            return [(b["ev"], acc.at[b["ridx"]])]

        issue(load_pairs(0, bufa), sem_ld_a)

        def pipe_body(i, carry):
            drain(load_pairs(2 * i, bufa), sem_ld_a)
            issue(gather_pairs(bufa), sem_g_a)

            @pl.when(i > 0)
            def _():
                drain(scat_pair(bufb), sem_sc_b)

            issue(load_pairs(2 * i + 1, bufb), sem_ld_b)
            drain(gather_pairs(bufa), sem_g_a)
            compute(bufa)
            issue(scat_pair(bufa), sem_sc_a, add=True)

            drain(load_pairs(2 * i + 1, bufb), sem_ld_b)
            issue(gather_pairs(bufb), sem_g_b)
            drain(scat_pair(bufa), sem_sc_a)

            @pl.when(i < _CHUNKS_PER_W // 2 - 1)
            def _():
                issue(load_pairs(2 * i + 2, bufa), sem_ld_a)

            drain(gather_pairs(bufb), sem_g_b)
            compute(bufb)
            issue(scat_pair(bufb), sem_sc_b, add=True)
            return carry

        lax.fori_loop(0, _CHUNKS_PER_W // 2, pipe_body, 0)
        drain(scat_pair(bufb), sem_sc_b)

        plsc.subcore_barrier()
        pltpu.sync_copy(acc.at[seg], out.at[cid, seg])

    return k(snd, rcv, shx, shy, shz, px, py, pz, zeros)


def _combine(partials):
    def body(p_ref, o_ref):
        o_ref[...] = p_ref[0, :] + p_ref[1, :]

    return pl.pallas_call(
        body,
        out_shape=jax.ShapeDtypeStruct((_NPAD,), jnp.float32),
    )(partials)


def kernel(positions, edge_index, shifts):
    sender = edge_index[0]
    receiver = edge_index[1]
    pad_e = _EPAD - _E
    snd = jnp.concatenate([sender, jnp.zeros((pad_e,), jnp.int32)])
    rcv = jnp.concatenate([receiver, jnp.full((pad_e,), _N, jnp.int32)])
    zpad_e = jnp.zeros((pad_e,), jnp.float32)
    st = shifts.T
    shx = jnp.concatenate([st[0], zpad_e])
    shy = jnp.concatenate([st[1], zpad_e])
    shz = jnp.concatenate([st[2], zpad_e])
    pt = positions.T
    zpad_n = jnp.zeros((_NPAD - _N,), jnp.float32)
    px = jnp.concatenate([pt[0], zpad_n])
    py = jnp.concatenate([pt[1], zpad_n])
    pz = jnp.concatenate([pt[2], zpad_n])
    zeros = jnp.zeros((_NPAD,), jnp.float32)
    partials = _sc_energies(snd, rcv, shx, shy, shz, px, py, pz, zeros)
    return _combine(partials)[:_N]
